# Initial kernel scaffold; baseline (speedup 1.0000x reference)
import jax
import jax.numpy as jnp
from jax.experimental import pallas as pl


def kernel(x, enc_wih_t, enc_b, enc_whh_t, dec_wih_t, dec_whh_t, dec_b):
    raise NotImplementedError("write your pallas kernel here")


# speedup vs baseline: 1.0813x; 1.0813x over previous
import jax
import jax.numpy as jnp
from jax.experimental import pallas as pl
from jax.experimental.pallas import tpu as pltpu


def _lstm_ae_kernel(xt_ref, wih_e_ref, b_e_ref, whh_e_ref,
                    wih_d_ref, whh_d_ref, b_d_ref, out_ref):
    T, Bt, I = xt_ref.shape
    H = whh_e_ref.shape[0]
    f32 = jnp.float32
    bf16 = jnp.bfloat16

    xw = jnp.dot(xt_ref[...].reshape(T * Bt, I), wih_e_ref[...],
                 preferred_element_type=f32) + b_e_ref[...]
    xw = xw.reshape(T, Bt, 4 * H)

    whh_e = whh_e_ref[...]

    h = jnp.zeros((Bt, H), f32)
    c = jnp.zeros((Bt, H), f32)
    for t in range(T):
        gates = xw[t] + jnp.dot(h.astype(bf16), whh_e,
                                preferred_element_type=f32)
        sig = jnp.tanh(gates[:, :3 * H]) * 0.5 + 0.5
        g_g = jnp.tanh(gates[:, 3 * H:] )
        i_g = sig[:, 0 * H:1 * H]
        f_g = sig[:, 1 * H:2 * H]
        o_g = sig[:, 2 * H:3 * H]
        c = f_g * c + i_g * g_g
        h = o_g * jnp.tanh(c)

    xw_d = jnp.dot(h.astype(bf16), wih_d_ref[...],
                   preferred_element_type=f32) + b_d_ref[...]
    whh_d = whh_d_ref[...]

    hd = jnp.zeros((Bt, I), f32)
    cd = jnp.zeros((Bt, I), f32)
    for t in range(T):
        gates = xw_d + jnp.dot(hd.astype(bf16), whh_d,
                               preferred_element_type=f32)
        sig = jnp.tanh(gates[:, :3 * I]) * 0.5 + 0.5
        g_g = jnp.tanh(gates[:, 3 * I:])
        i_g = sig[:, 0 * I:1 * I]
        f_g = sig[:, 1 * I:2 * I]
        o_g = sig[:, 2 * I:3 * I]
        cd = f_g * cd + i_g * g_g
        hd = o_g * jnp.tanh(cd)
        out_ref[:, t * I:(t + 1) * I] = hd


def _halve_sig_cols(w, n):
    return jnp.concatenate([w[..., :3 * n] * 0.5, w[..., 3 * n:]], axis=-1)


@jax.jit
def _forward(x, enc_wih_t, enc_b, enc_whh_t, dec_wih_t, dec_whh_t, dec_b):
    B, T, I = x.shape
    H = enc_whh_t.shape[0]
    bf16 = jnp.bfloat16

    xt = jnp.transpose(x, (1, 0, 2)).astype(bf16)
    wih_e = _halve_sig_cols(enc_wih_t, H).astype(bf16)
    whh_e = _halve_sig_cols(enc_whh_t, H).astype(bf16)
    b_e = _halve_sig_cols(enc_b, H)
    wih_d = _halve_sig_cols(dec_wih_t, I).astype(bf16)
    whh_d = _halve_sig_cols(dec_whh_t, I).astype(bf16)
    b_d = _halve_sig_cols(dec_b, I)

    bt = B // 2 if (B % 16 == 0) else B
    grid = (B // bt,)

    out_flat = pl.pallas_call(
        _lstm_ae_kernel,
        out_shape=jax.ShapeDtypeStruct((B, T * I), jnp.float32),
        grid=grid,
        in_specs=[
            pl.BlockSpec((T, bt, I), lambda b: (0, b, 0)),
            pl.BlockSpec((I, 4 * H), lambda b: (0, 0)),
            pl.BlockSpec((1, 4 * H), lambda b: (0, 0)),
            pl.BlockSpec((H, 4 * H), lambda b: (0, 0)),
            pl.BlockSpec((H, 4 * I), lambda b: (0, 0)),
            pl.BlockSpec((I, 4 * I), lambda b: (0, 0)),
            pl.BlockSpec((1, 4 * I), lambda b: (0, 0)),
        ],
        out_specs=pl.BlockSpec((bt, T * I), lambda b: (b, 0)),
        compiler_params=pltpu.CompilerParams(
            dimension_semantics=("parallel",),
            vmem_limit_bytes=48 * 1024 * 1024),
    )(xt, wih_e, b_e, whh_e, wih_d, whh_d, b_d)

    return out_flat.reshape(B, T, I)


def kernel(x, enc_wih_t, enc_b, enc_whh_t, dec_wih_t, dec_whh_t, dec_b):
    return _forward(x, enc_wih_t, enc_b, enc_whh_t, dec_wih_t,
                    dec_whh_t, dec_b)
